# baseline (speedup 1.0000x reference)
import functools
import jax
import jax.numpy as jnp
from jax import lax
from jax.experimental import pallas as pl
from jax.experimental.pallas import tpu as pltpu
from jax.experimental.pallas import tpu_sc as plsc

NC = 2
NS = 16
NW = NC * NS


def _emb_gather(emb, xpad, n_pad, d):
  rows_w = n_pad // NW
  ch = 80
  nch = rows_w // ch
  mesh = plsc.VectorSubcoreMesh(core_axis_name="c", subcore_axis_name="s")

  @functools.partial(
      pl.kernel,
      out_type=jax.ShapeDtypeStruct((n_pad, d), jnp.float32),
      mesh=mesh,
      scratch_types=[
          pltpu.VMEM((ch,), jnp.int32),
          pltpu.VMEM((ch, d), jnp.float32),
          pltpu.SemaphoreType.DMA,
      ],
  )
  def gather_k(emb_hbm, x_hbm, out_hbm, idx_v, rows_v, sem):
    wid = lax.axis_index("s") * NC + lax.axis_index("c")
    base = wid * rows_w

    def body(j, carry):
      off = base + j * ch
      pltpu.sync_copy(x_hbm.at[pl.ds(off, ch)], idx_v)
      pltpu.async_copy(emb_hbm.at[idx_v], rows_v, sem).wait()
      pltpu.sync_copy(rows_v, out_hbm.at[pl.ds(off, ch)])
      return carry

    lax.fori_loop(0, nch, body, 0)

  return gather_k(emb, xpad)


def _edge_pass(K, QV, idxcat, n_pad, d, e_pad):
  c = 40
  nch_tot = e_pad // (NS * c)
  nf = 356
  ns = nch_tot - nf
  fast_cid = 0
  rows_acc = n_pad // NS
  mesh = plsc.VectorSubcoreMesh(core_axis_name="c", subcore_axis_name="s")
  zeros = jnp.zeros((rows_acc, d), jnp.float32)

  @functools.partial(
      pl.kernel,
      out_type=jax.ShapeDtypeStruct((NC, n_pad, d), jnp.float32),
      mesh=mesh,
      scratch_types=[
          pltpu.VMEM((4, 3, c), jnp.int32),
          pltpu.VMEM((2, c, d), jnp.float32),
          pltpu.VMEM((2, c, 2 * d), jnp.float32),
          pltpu.VMEM((2, c, d), jnp.float32),
          pltpu.VMEM_SHARED((n_pad, d), jnp.float32),
          pltpu.SemaphoreType.DMA,
          pltpu.SemaphoreType.DMA,
          pltpu.SemaphoreType.DMA,
          pltpu.SemaphoreType.DMA,
          pltpu.SemaphoreType.DMA,
          pltpu.SemaphoreType.DMA,
          pltpu.SemaphoreType.DMA,
      ],
  )
  def edge_k(k_hbm, qv_hbm, idx_hbm, z_hbm, out_hbm,
             ib, kv, qv, mv, acc,
             sk0, sk1, sq0, sq1, ss0, ss1, si):
    cid = lax.axis_index("c")
    sid = lax.axis_index("s")
    is_fast = cid == fast_cid
    nch = jnp.where(is_fast, nf, ns)
    row0 = jnp.where(is_fast, sid * nf, NS * nf + sid * ns)
    sks = (sk0, sk1)
    sqs = (sq0, sq1)
    sss = (ss0, ss1)

    pltpu.sync_copy(z_hbm, acc.at[pl.ds(sid * rows_acc, rows_acc)])
    plsc.subcore_barrier()

    def fire(j, u, b):
      pltpu.async_copy(k_hbm.at[ib.at[u, 1]], kv.at[b], sks[b])
      pltpu.async_copy(qv_hbm.at[ib.at[u, 0]], qv.at[b], sqs[b])

    for b in range(2):
      pltpu.sync_copy(idx_hbm.at[row0 + b], ib.at[b])
      fire(b, b, b)

    def quad(jo, carry):
      for u in range(4):
        b = u % 2
        u2 = (u + 2) % 4
        j = jo * 4 + u
        pltpu.make_async_copy(k_hbm.at[ib.at[u, 1]], kv.at[b], sks[b]).wait()
        pltpu.make_async_copy(qv_hbm.at[ib.at[u, 0]], qv.at[b], sqs[b]).wait()

        @pl.when(j >= 2)
        def _():
          pltpu.make_async_copy(mv.at[b], acc.at[ib.at[u2, 2]], sss[b]).wait()

        @pl.when(j + 2 < nch)
        def _():
          pltpu.async_copy(idx_hbm.at[row0 + j + 2], ib.at[u2], si)

        def rowp(r2, rc):
          for rr in range(2):
            r = r2 * 2 + rr
            for cb in range(d // 16):
              sl = pl.ds(cb * 16, 16)
              slv = pl.ds(d + cb * 16, 16)
              g = kv[b, r, sl] + qv[b, r, sl]
              sig = 1.0 / (1.0 + jnp.exp(g))
              mv[b, r, sl] = sig * qv[b, r, slv]
          return rc
        lax.fori_loop(0, c // 2, rowp, 0)

        pltpu.async_copy(mv.at[b], acc.at[ib.at[u, 2]], sss[b], add=True)

        @pl.when(j + 2 < nch)
        def _():
          pltpu.make_async_copy(idx_hbm.at[row0 + j + 2], ib.at[u2], si).wait()
          fire(j + 2, u2, b)
      return carry

    lax.fori_loop(0, nch // 4, quad, 0)

    for b in range(2):
      pltpu.make_async_copy(mv.at[b], acc.at[ib.at[2 + b, 2]], sss[b]).wait()

    plsc.subcore_barrier()
    pltpu.sync_copy(acc.at[pl.ds(sid * rows_acc, rows_acc)],
                    out_hbm.at[cid, pl.ds(sid * rows_acc, rows_acc)])

  return edge_k(K, QV, idxcat, zeros)


def _projections(h, wkT, bk, wqvT, bqv, wsT, bo):
  n, d = h.shape
  br = 2000
  grid = n // br

  def body(h_ref, wk_r, bk_r, wqv_r, bqv_r, ws_r, bo_r, k_o, qv_o, s_o):
    hh = h_ref[...]
    k_o[...] = jnp.dot(hh, wk_r[...], preferred_element_type=jnp.float32) + bk_r[...]
    qv_o[...] = jnp.dot(hh, wqv_r[...], preferred_element_type=jnp.float32) + bqv_r[...]
    s_o[...] = jnp.dot(hh, ws_r[...], preferred_element_type=jnp.float32) + bo_r[...]

  row_spec = pl.BlockSpec((br, d), lambda i: (i, 0))
  row2_spec = pl.BlockSpec((br, 2 * d), lambda i: (i, 0))
  w_spec = pl.BlockSpec((d, d), lambda i: (0, 0))
  w2_spec = pl.BlockSpec((d, 2 * d), lambda i: (0, 0))
  b_spec = pl.BlockSpec((1, d), lambda i: (0, 0))
  b2_spec = pl.BlockSpec((1, 2 * d), lambda i: (0, 0))
  return pl.pallas_call(
      body,
      grid=(grid,),
      in_specs=[row_spec, w_spec, b_spec, w2_spec, b2_spec, w_spec, b_spec],
      out_specs=[row_spec, row2_spec, row_spec],
      out_shape=[jax.ShapeDtypeStruct((n, d), jnp.float32),
                 jax.ShapeDtypeStruct((n, 2 * d), jnp.float32),
                 jax.ShapeDtypeStruct((n, d), jnp.float32)],
  )(h, wkT, bk.reshape(1, d), wqvT, bqv.reshape(1, 2 * d),
    wsT, bo.reshape(1, d))


def kernel(x, edge_index, emb, Wk1, bk1, Wq1, bq1, Wv1, bv1, Ws1, bo1,
           gamma1, beta1, Wk2, bk2, Wq2, bq2, Wv2, bv2, Ws2, bo2,
           gamma2, beta2, fcW, fcb):
  n, d = emb.shape
  e = edge_index.shape[1]
  n_pad = 10240
  n_acc = 10112
  e_pad = 327680

  x = x.astype(jnp.int32)
  src = edge_index[0].astype(jnp.int32)
  dst = edge_index[1].astype(jnp.int32)
  xpad = jnp.concatenate([x, jnp.zeros((n_pad - n,), jnp.int32)])
  srcp = jnp.concatenate([src, jnp.zeros((e_pad - e,), jnp.int32)])
  dstg = jnp.concatenate([dst, jnp.zeros((e_pad - e,), jnp.int32)])
  dsts = jnp.concatenate([dst, jnp.full((e_pad - e,), n, jnp.int32)])
  cc = 40
  idxcat = jnp.stack([srcp.reshape(-1, cc), dstg.reshape(-1, cc),
                      dsts.reshape(-1, cc)], axis=1)

  h = _emb_gather(emb, xpad, n_pad, d)[:n]

  neg = jnp.float32(-1.0)
  wqv1 = jnp.concatenate([Wq1.T * neg, Wv1.T], axis=1)
  bqv1 = jnp.concatenate([bq1 * neg, bv1])
  wqv2 = jnp.concatenate([Wq2.T * neg, Wv2.T], axis=1)
  bqv2 = jnp.concatenate([bq2 * neg, bv2])
  k1, qv1, s1 = _projections(h, Wk1.T * neg, bk1 * neg, wqv1, bqv1,
                             Ws1.T, bo1)
  p1 = _edge_pass(k1, qv1, idxcat, n_acc, d, e_pad)
  k2, qv2, s2 = _combine_bn_proj(p1, s1, gamma1, beta1,
                                 Wk2.T * neg, bk2 * neg, wqv2, bqv2,
                                 Ws2.T, bo2)
  p2 = _edge_pass(k2, qv2, idxcat, n_acc, d, e_pad)
  return _final_affine(p2, s2, gamma2, beta2, fcW.T, fcb)


def _combine_bn_proj(p, s, gamma, beta, wkT, bk, wqvT, bqv, wsT, bo):
  n, d = s.shape

  def body(p_ref, s_ref, g_ref, b_ref, wk_r, bk_r, wqv_r, bqv_r,
           ws_r, bo_r, k_o, qv_o, s_o):
    pre = p_ref[0, :n, :] + p_ref[1, :n, :] + s_ref[...]
    mu = jnp.sum(pre, axis=0, keepdims=True) * (1.0 / n)
    cen = pre - mu
    var = jnp.sum(cen * cen, axis=0, keepdims=True) * (1.0 / n)
    h = jnp.maximum(
        cen * lax.rsqrt(var + 1e-5) * g_ref[...] + b_ref[...], 0.0)
    k_o[...] = jnp.dot(h, wk_r[...], preferred_element_type=jnp.float32) + bk_r[...]
    qv_o[...] = jnp.dot(h, wqv_r[...], preferred_element_type=jnp.float32) + bqv_r[...]
    s_o[...] = jnp.dot(h, ws_r[...], preferred_element_type=jnp.float32) + bo_r[...]

  return pl.pallas_call(
      body,
      out_shape=[jax.ShapeDtypeStruct((n, d), jnp.float32),
                 jax.ShapeDtypeStruct((n, 2 * d), jnp.float32),
                 jax.ShapeDtypeStruct((n, d), jnp.float32)],
  )(p, s, gamma.reshape(1, d), beta.reshape(1, d), wkT, bk.reshape(1, d),
    wqvT, bqv.reshape(1, 2 * d), wsT, bo.reshape(1, d))


def _final_affine(p, s, gamma, beta, fcT, fcb):
  n, d = s.shape

  def body(p_ref, s_ref, g_ref, b_ref, w_ref, fb_ref, o_ref):
    pre = p_ref[0, :n, :] + p_ref[1, :n, :] + s_ref[...]
    mu = jnp.sum(pre, axis=0, keepdims=True) * (1.0 / n)
    cen = pre - mu
    var = jnp.sum(cen * cen, axis=0, keepdims=True) * (1.0 / n)
    h = jnp.maximum(
        cen * lax.rsqrt(var + 1e-5) * g_ref[...] + b_ref[...], 0.0)
    o_ref[...] = jnp.dot(h, w_ref[...], preferred_element_type=jnp.float32) + fb_ref[...]

  return pl.pallas_call(
      body,
      out_shape=jax.ShapeDtypeStruct((n, d), jnp.float32),
  )(p, s, gamma.reshape(1, d), beta.reshape(1, d), fcT, fcb.reshape(1, d))


# speedup vs baseline: 3.3153x; 3.3153x over previous
import functools
import jax
import jax.numpy as jnp
from jax import lax
from jax.experimental import pallas as pl
from jax.experimental.pallas import tpu as pltpu
from jax.experimental.pallas import tpu_sc as plsc

NC = 2
NS = 16
NW = NC * NS


def _emb_gather(emb, xpad, n_pad, d):
  rows_w = n_pad // NW
  ch = 80
  nch = rows_w // ch
  mesh = plsc.VectorSubcoreMesh(core_axis_name="c", subcore_axis_name="s")

  @functools.partial(
      pl.kernel,
      out_type=jax.ShapeDtypeStruct((n_pad, d), jnp.float32),
      mesh=mesh,
      scratch_types=[
          pltpu.VMEM((ch,), jnp.int32),
          pltpu.VMEM((ch, d), jnp.float32),
          pltpu.SemaphoreType.DMA,
      ],
  )
  def gather_k(emb_hbm, x_hbm, out_hbm, idx_v, rows_v, sem):
    wid = lax.axis_index("s") * NC + lax.axis_index("c")
    base = wid * rows_w

    def body(j, carry):
      off = base + j * ch
      pltpu.sync_copy(x_hbm.at[pl.ds(off, ch)], idx_v)
      pltpu.async_copy(emb_hbm.at[idx_v], rows_v, sem).wait()
      pltpu.sync_copy(rows_v, out_hbm.at[pl.ds(off, ch)])
      return carry

    lax.fori_loop(0, nch, body, 0)

  return gather_k(emb, xpad)


def _edge_pass(K, Q, V, idxcat, n_pad, d, e_pad):
  c = 40
  nch_tot = e_pad // (NS * c)
  nf = 356
  ns = nch_tot - nf
  fast_cid = 0
  rows_acc = n_pad // NS
  mesh = plsc.VectorSubcoreMesh(core_axis_name="c", subcore_axis_name="s")
  zeros = jnp.zeros((rows_acc, d), jnp.float32)

  @functools.partial(
      pl.kernel,
      out_type=jax.ShapeDtypeStruct((NC, n_pad, d), jnp.float32),
      mesh=mesh,
      scratch_types=[
          pltpu.VMEM((4, 3, c), jnp.int32),
          pltpu.VMEM((2, c, d), jnp.float32),
          pltpu.VMEM((2, c, d), jnp.float32),
          pltpu.VMEM((2, c, d), jnp.float32),
          pltpu.VMEM((2, c, d), jnp.float32),
          pltpu.VMEM_SHARED((n_pad, d), jnp.float32),
          pltpu.SemaphoreType.DMA,
          pltpu.SemaphoreType.DMA,
          pltpu.SemaphoreType.DMA,
          pltpu.SemaphoreType.DMA,
          pltpu.SemaphoreType.DMA,
          pltpu.SemaphoreType.DMA,
          pltpu.SemaphoreType.DMA,
          pltpu.SemaphoreType.DMA,
          pltpu.SemaphoreType.DMA,
      ],
  )
  def edge_k(k_hbm, q_hbm, v_hbm, idx_hbm, z_hbm, out_hbm,
             ib, kv, qv, vv, mv, acc,
             sk0, sk1, sq0, sq1, sv0, sv1, ss0, ss1, si):
    cid = lax.axis_index("c")
    sid = lax.axis_index("s")
    is_fast = cid == fast_cid
    nch = jnp.where(is_fast, nf, ns)
    row0 = jnp.where(is_fast, sid * nf, NS * nf + sid * ns)
    sks = (sk0, sk1)
    sqs = (sq0, sq1)
    svs = (sv0, sv1)
    sss = (ss0, ss1)

    pltpu.sync_copy(z_hbm, acc.at[pl.ds(sid * rows_acc, rows_acc)])
    plsc.subcore_barrier()

    def fire(j, u, b):
      pltpu.async_copy(k_hbm.at[ib.at[u, 1]], kv.at[b], sks[b])
      pltpu.async_copy(q_hbm.at[ib.at[u, 0]], qv.at[b], sqs[b])
      pltpu.async_copy(v_hbm.at[ib.at[u, 0]], vv.at[b], svs[b])

    for b in range(2):
      pltpu.sync_copy(idx_hbm.at[row0 + b], ib.at[b])
      fire(b, b, b)

    def quad(jo, carry):
      for u in range(4):
        b = u % 2
        u2 = (u + 2) % 4
        j = jo * 4 + u
        pltpu.make_async_copy(k_hbm.at[ib.at[u, 1]], kv.at[b], sks[b]).wait()
        pltpu.make_async_copy(q_hbm.at[ib.at[u, 0]], qv.at[b], sqs[b]).wait()
        pltpu.make_async_copy(v_hbm.at[ib.at[u, 0]], vv.at[b], svs[b]).wait()

        @pl.when(j >= 2)
        def _():
          pltpu.make_async_copy(mv.at[b], acc.at[ib.at[u2, 2]], sss[b]).wait()

        @pl.when(j + 2 < nch)
        def _():
          pltpu.async_copy(idx_hbm.at[row0 + j + 2], ib.at[u2], si)

        def rowp(r2, rc):
          for rr in range(2):
            r = r2 * 2 + rr
            for cb in range(d // 16):
              sl = pl.ds(cb * 16, 16)
              g = kv[b, r, sl] + qv[b, r, sl]
              sig = 1.0 / (1.0 + jnp.exp(g))
              mv[b, r, sl] = sig * vv[b, r, sl]
          return rc
        lax.fori_loop(0, c // 2, rowp, 0)

        pltpu.async_copy(mv.at[b], acc.at[ib.at[u, 2]], sss[b], add=True)

        @pl.when(j + 2 < nch)
        def _():
          pltpu.make_async_copy(idx_hbm.at[row0 + j + 2], ib.at[u2], si).wait()
          fire(j + 2, u2, b)
      return carry

    lax.fori_loop(0, nch // 4, quad, 0)

    for b in range(2):
      pltpu.make_async_copy(mv.at[b], acc.at[ib.at[2 + b, 2]], sss[b]).wait()

    plsc.subcore_barrier()
    pltpu.sync_copy(acc.at[pl.ds(sid * rows_acc, rows_acc)],
                    out_hbm.at[cid, pl.ds(sid * rows_acc, rows_acc)])

  return edge_k(K, Q, V, idxcat, zeros)


def _projections(h, wkT, bk, wqT, bq, wvT, bv, wsT, bo):
  n, d = h.shape

  def body(h_ref, wk_r, bk_r, wq_r, bq_r, wv_r, bv_r, ws_r, bo_r,
           k_o, q_o, v_o, s_o):
    hh = h_ref[...]
    k_o[...] = jnp.dot(hh, wk_r[...], preferred_element_type=jnp.float32) + bk_r[...]
    q_o[...] = jnp.dot(hh, wq_r[...], preferred_element_type=jnp.float32) + bq_r[...]
    v_o[...] = jnp.dot(hh, wv_r[...], preferred_element_type=jnp.float32) + bv_r[...]
    s_o[...] = jnp.dot(hh, ws_r[...], preferred_element_type=jnp.float32) + bo_r[...]

  out = jax.ShapeDtypeStruct((n, d), jnp.float32)
  return pl.pallas_call(
      body,
      out_shape=[out, out, out, out],
  )(h, wkT, bk.reshape(1, d), wqT, bq.reshape(1, d), wvT, bv.reshape(1, d),
    wsT, bo.reshape(1, d))


def kernel(x, edge_index, emb, Wk1, bk1, Wq1, bq1, Wv1, bv1, Ws1, bo1,
           gamma1, beta1, Wk2, bk2, Wq2, bq2, Wv2, bv2, Ws2, bo2,
           gamma2, beta2, fcW, fcb):
  n, d = emb.shape
  e = edge_index.shape[1]
  n_pad = 10240
  n_acc = 10112
  e_pad = 327680

  x = x.astype(jnp.int32)
  src = edge_index[0].astype(jnp.int32)
  dst = edge_index[1].astype(jnp.int32)
  xpad = jnp.concatenate([x, jnp.zeros((n_pad - n,), jnp.int32)])
  srcp = jnp.concatenate([src, jnp.zeros((e_pad - e,), jnp.int32)])
  dstg = jnp.concatenate([dst, jnp.zeros((e_pad - e,), jnp.int32)])
  dsts = jnp.concatenate([dst, jnp.full((e_pad - e,), n, jnp.int32)])
  cc = 40
  idxcat = jnp.stack([srcp.reshape(-1, cc), dstg.reshape(-1, cc),
                      dsts.reshape(-1, cc)], axis=1)

  h = _emb_gather(emb, xpad, n_pad, d)[:n]

  neg = jnp.float32(-1.0)
  k1, q1, v1, s1 = _projections(h, Wk1.T * neg, bk1 * neg,
                                Wq1.T * neg, bq1 * neg, Wv1.T, bv1,
                                Ws1.T, bo1)
  p1 = _edge_pass(k1, q1, v1, idxcat, n_acc, d, e_pad)
  k2, q2, v2, s2 = _combine_bn_proj(p1, s1, gamma1, beta1,
                                    Wk2.T * neg, bk2 * neg,
                                    Wq2.T * neg, bq2 * neg, Wv2.T, bv2,
                                    Ws2.T, bo2)
  p2 = _edge_pass(k2, q2, v2, idxcat, n_acc, d, e_pad)
  return _final_affine(p2, s2, gamma2, beta2, fcW.T, fcb)


def _combine_bn_proj(p, s, gamma, beta, wkT, bk, wqT, bq, wvT, bv, wsT, bo):
  n, d = s.shape

  def body(p_ref, s_ref, g_ref, b_ref, wk_r, bk_r, wq_r, bq_r, wv_r, bv_r,
           ws_r, bo_r, k_o, q_o, v_o, s_o):
    pre = p_ref[0, :n, :] + p_ref[1, :n, :] + s_ref[...]
    mu = jnp.sum(pre, axis=0, keepdims=True) * (1.0 / n)
    cen = pre - mu
    var = jnp.sum(cen * cen, axis=0, keepdims=True) * (1.0 / n)
    h = jnp.maximum(
        cen * lax.rsqrt(var + 1e-5) * g_ref[...] + b_ref[...], 0.0)
    k_o[...] = jnp.dot(h, wk_r[...], preferred_element_type=jnp.float32) + bk_r[...]
    q_o[...] = jnp.dot(h, wq_r[...], preferred_element_type=jnp.float32) + bq_r[...]
    v_o[...] = jnp.dot(h, wv_r[...], preferred_element_type=jnp.float32) + bv_r[...]
    s_o[...] = jnp.dot(h, ws_r[...], preferred_element_type=jnp.float32) + bo_r[...]

  out = jax.ShapeDtypeStruct((n, d), jnp.float32)
  return pl.pallas_call(
      body,
      out_shape=[out, out, out, out],
  )(p, s, gamma.reshape(1, d), beta.reshape(1, d), wkT, bk.reshape(1, d),
    wqT, bq.reshape(1, d), wvT, bv.reshape(1, d), wsT, bo.reshape(1, d))


def _final_affine(p, s, gamma, beta, fcT, fcb):
  n, d = s.shape

  def body(p_ref, s_ref, g_ref, b_ref, w_ref, fb_ref, o_ref):
    pre = p_ref[0, :n, :] + p_ref[1, :n, :] + s_ref[...]
    mu = jnp.sum(pre, axis=0, keepdims=True) * (1.0 / n)
    cen = pre - mu
    var = jnp.sum(cen * cen, axis=0, keepdims=True) * (1.0 / n)
    h = jnp.maximum(
        cen * lax.rsqrt(var + 1e-5) * g_ref[...] + b_ref[...], 0.0)
    o_ref[...] = jnp.dot(h, w_ref[...], preferred_element_type=jnp.float32) + fb_ref[...]

  return pl.pallas_call(
      body,
      out_shape=jax.ShapeDtypeStruct((n, d), jnp.float32),
  )(p, s, gamma.reshape(1, d), beta.reshape(1, d), fcT, fcb.reshape(1, d))


# speedup vs baseline: 3.7983x; 1.1457x over previous
import functools
import jax
import jax.numpy as jnp
from jax import lax
from jax.experimental import pallas as pl
from jax.experimental.pallas import tpu as pltpu
from jax.experimental.pallas import tpu_sc as plsc

NC = 2
NS = 16
NW = NC * NS


def _emb_gather(emb, xpad, n_pad, d):
  rows_w = n_pad // NW
  ch = 80
  nch = rows_w // ch
  mesh = plsc.VectorSubcoreMesh(core_axis_name="c", subcore_axis_name="s")

  @functools.partial(
      pl.kernel,
      out_type=jax.ShapeDtypeStruct((n_pad, d), jnp.float32),
      mesh=mesh,
      scratch_types=[
          pltpu.VMEM((ch,), jnp.int32),
          pltpu.VMEM((ch, d), jnp.float32),
          pltpu.SemaphoreType.DMA,
      ],
  )
  def gather_k(emb_hbm, x_hbm, out_hbm, idx_v, rows_v, sem):
    wid = lax.axis_index("s") * NC + lax.axis_index("c")
    base = wid * rows_w

    def body(j, carry):
      off = base + j * ch
      pltpu.sync_copy(x_hbm.at[pl.ds(off, ch)], idx_v)
      pltpu.async_copy(emb_hbm.at[idx_v], rows_v, sem).wait()
      pltpu.sync_copy(rows_v, out_hbm.at[pl.ds(off, ch)])
      return carry

    lax.fori_loop(0, nch, body, 0)

  return gather_k(emb, xpad)


def _edge_pass(K, Q, V, idxcat, n_pad, d, e_pad):
  c = 40
  nch_tot = e_pad // (NS * c)
  nf = 416
  ns = nch_tot - nf
  fast_cid = 0
  rows_acc = n_pad // NS
  mesh = plsc.VectorSubcoreMesh(core_axis_name="c", subcore_axis_name="s")
  zeros = jnp.zeros((rows_acc, d), jnp.float32)

  @functools.partial(
      pl.kernel,
      out_type=jax.ShapeDtypeStruct((NC, n_pad, d), jnp.float32),
      mesh=mesh,
      scratch_types=[
          pltpu.VMEM((4, 3, c), jnp.int32),
          pltpu.VMEM((2, c, d), jnp.float32),
          pltpu.VMEM((2, c, d), jnp.float32),
          pltpu.VMEM((2, c, d), jnp.float32),
          pltpu.VMEM((2, c, d), jnp.float32),
          pltpu.VMEM_SHARED((n_pad, d), jnp.float32),
          pltpu.SemaphoreType.DMA,
          pltpu.SemaphoreType.DMA,
          pltpu.SemaphoreType.DMA,
          pltpu.SemaphoreType.DMA,
          pltpu.SemaphoreType.DMA,
          pltpu.SemaphoreType.DMA,
          pltpu.SemaphoreType.DMA,
          pltpu.SemaphoreType.DMA,
          pltpu.SemaphoreType.DMA,
      ],
  )
  def edge_k(k_hbm, q_hbm, v_hbm, idx_hbm, z_hbm, out_hbm,
             ib, kv, qv, vv, mv, acc,
             sk0, sk1, sq0, sq1, sv0, sv1, ss0, ss1, si):
    cid = lax.axis_index("c")
    sid = lax.axis_index("s")
    is_fast = cid == fast_cid
    nch = jnp.where(is_fast, nf, ns)
    row0 = jnp.where(is_fast, sid * nf, NS * nf + sid * ns)
    sks = (sk0, sk1)
    sqs = (sq0, sq1)
    svs = (sv0, sv1)
    sss = (ss0, ss1)

    pltpu.sync_copy(z_hbm, acc.at[pl.ds(sid * rows_acc, rows_acc)])
    plsc.subcore_barrier()

    def fire(j, u, b):
      pltpu.async_copy(k_hbm.at[ib.at[u, 1]], kv.at[b], sks[b])
      pltpu.async_copy(q_hbm.at[ib.at[u, 0]], qv.at[b], sqs[b])
      pltpu.async_copy(v_hbm.at[ib.at[u, 0]], vv.at[b], svs[b])

    for b in range(2):
      pltpu.sync_copy(idx_hbm.at[row0 + b], ib.at[b])
      fire(b, b, b)

    def quad(jo, carry):
      for u in range(4):
        b = u % 2
        u2 = (u + 2) % 4
        j = jo * 4 + u
        pltpu.make_async_copy(k_hbm.at[ib.at[u, 1]], kv.at[b], sks[b]).wait()
        pltpu.make_async_copy(q_hbm.at[ib.at[u, 0]], qv.at[b], sqs[b]).wait()
        pltpu.make_async_copy(v_hbm.at[ib.at[u, 0]], vv.at[b], svs[b]).wait()

        @pl.when(j >= 2)
        def _():
          pltpu.make_async_copy(mv.at[b], acc.at[ib.at[u2, 2]], sss[b]).wait()

        @pl.when(j + 2 < nch)
        def _():
          pltpu.async_copy(idx_hbm.at[row0 + j + 2], ib.at[u2], si)

        def rowp(r2, rc):
          for rr in range(2):
            r = r2 * 2 + rr
            for cb in range(d // 16):
              sl = pl.ds(cb * 16, 16)
              g = kv[b, r, sl] + qv[b, r, sl]
              sig = 1.0 / (1.0 + jnp.exp(g))
              mv[b, r, sl] = sig * vv[b, r, sl]
          return rc
        lax.fori_loop(0, c // 2, rowp, 0)

        pltpu.async_copy(mv.at[b], acc.at[ib.at[u, 2]], sss[b], add=True)

        @pl.when(j + 2 < nch)
        def _():
          pltpu.make_async_copy(idx_hbm.at[row0 + j + 2], ib.at[u2], si).wait()
          fire(j + 2, u2, b)
      return carry

    lax.fori_loop(0, nch // 4, quad, 0)

    for b in range(2):
      pltpu.make_async_copy(mv.at[b], acc.at[ib.at[2 + b, 2]], sss[b]).wait()

    plsc.subcore_barrier()
    pltpu.sync_copy(acc.at[pl.ds(sid * rows_acc, rows_acc)],
                    out_hbm.at[cid, pl.ds(sid * rows_acc, rows_acc)])

  return edge_k(K, Q, V, idxcat, zeros)


def _projections(h, wkT, bk, wqT, bq, wvT, bv, wsT, bo):
  n, d = h.shape

  def body(h_ref, wk_r, bk_r, wq_r, bq_r, wv_r, bv_r, ws_r, bo_r,
           k_o, q_o, v_o, s_o):
    hh = h_ref[...]
    k_o[...] = jnp.dot(hh, wk_r[...], preferred_element_type=jnp.float32) + bk_r[...]
    q_o[...] = jnp.dot(hh, wq_r[...], preferred_element_type=jnp.float32) + bq_r[...]
    v_o[...] = jnp.dot(hh, wv_r[...], preferred_element_type=jnp.float32) + bv_r[...]
    s_o[...] = jnp.dot(hh, ws_r[...], preferred_element_type=jnp.float32) + bo_r[...]

  out = jax.ShapeDtypeStruct((n, d), jnp.float32)
  return pl.pallas_call(
      body,
      out_shape=[out, out, out, out],
  )(h, wkT, bk.reshape(1, d), wqT, bq.reshape(1, d), wvT, bv.reshape(1, d),
    wsT, bo.reshape(1, d))


def kernel(x, edge_index, emb, Wk1, bk1, Wq1, bq1, Wv1, bv1, Ws1, bo1,
           gamma1, beta1, Wk2, bk2, Wq2, bq2, Wv2, bv2, Ws2, bo2,
           gamma2, beta2, fcW, fcb):
  n, d = emb.shape
  e = edge_index.shape[1]
  n_pad = 10240
  n_acc = 10112
  e_pad = 327680

  x = x.astype(jnp.int32)
  src = edge_index[0].astype(jnp.int32)
  dst = edge_index[1].astype(jnp.int32)
  xpad = jnp.concatenate([x, jnp.zeros((n_pad - n,), jnp.int32)])
  srcp = jnp.concatenate([src, jnp.zeros((e_pad - e,), jnp.int32)])
  dstg = jnp.concatenate([dst, jnp.zeros((e_pad - e,), jnp.int32)])
  dsts = jnp.concatenate([dst, jnp.full((e_pad - e,), n, jnp.int32)])
  cc = 40
  idxcat = jnp.stack([srcp.reshape(-1, cc), dstg.reshape(-1, cc),
                      dsts.reshape(-1, cc)], axis=1)

  h = _emb_gather(emb, xpad, n_pad, d)[:n]

  neg = jnp.float32(-1.0)
  k1, q1, v1, s1 = _projections(h, Wk1.T * neg, bk1 * neg,
                                Wq1.T * neg, bq1 * neg, Wv1.T, bv1,
                                Ws1.T, bo1)
  p1 = _edge_pass(k1, q1, v1, idxcat, n_acc, d, e_pad)
  k2, q2, v2, s2 = _combine_bn_proj(p1, s1, gamma1, beta1,
                                    Wk2.T * neg, bk2 * neg,
                                    Wq2.T * neg, bq2 * neg, Wv2.T, bv2,
                                    Ws2.T, bo2)
  p2 = _edge_pass(k2, q2, v2, idxcat, n_acc, d, e_pad)
  return _final_affine(p2, s2, gamma2, beta2, fcW.T, fcb)


def _combine_bn_proj(p, s, gamma, beta, wkT, bk, wqT, bq, wvT, bv, wsT, bo):
  n, d = s.shape

  def body(p_ref, s_ref, g_ref, b_ref, wk_r, bk_r, wq_r, bq_r, wv_r, bv_r,
           ws_r, bo_r, k_o, q_o, v_o, s_o):
    pre = p_ref[0, :n, :] + p_ref[1, :n, :] + s_ref[...]
    mu = jnp.sum(pre, axis=0, keepdims=True) * (1.0 / n)
    cen = pre - mu
    var = jnp.sum(cen * cen, axis=0, keepdims=True) * (1.0 / n)
    h = jnp.maximum(
        cen * lax.rsqrt(var + 1e-5) * g_ref[...] + b_ref[...], 0.0)
    k_o[...] = jnp.dot(h, wk_r[...], preferred_element_type=jnp.float32) + bk_r[...]
    q_o[...] = jnp.dot(h, wq_r[...], preferred_element_type=jnp.float32) + bq_r[...]
    v_o[...] = jnp.dot(h, wv_r[...], preferred_element_type=jnp.float32) + bv_r[...]
    s_o[...] = jnp.dot(h, ws_r[...], preferred_element_type=jnp.float32) + bo_r[...]

  out = jax.ShapeDtypeStruct((n, d), jnp.float32)
  return pl.pallas_call(
      body,
      out_shape=[out, out, out, out],
  )(p, s, gamma.reshape(1, d), beta.reshape(1, d), wkT, bk.reshape(1, d),
    wqT, bq.reshape(1, d), wvT, bv.reshape(1, d), wsT, bo.reshape(1, d))


def _final_affine(p, s, gamma, beta, fcT, fcb):
  n, d = s.shape

  def body(p_ref, s_ref, g_ref, b_ref, w_ref, fb_ref, o_ref):
    pre = p_ref[0, :n, :] + p_ref[1, :n, :] + s_ref[...]
    mu = jnp.sum(pre, axis=0, keepdims=True) * (1.0 / n)
    cen = pre - mu
    var = jnp.sum(cen * cen, axis=0, keepdims=True) * (1.0 / n)
    h = jnp.maximum(
        cen * lax.rsqrt(var + 1e-5) * g_ref[...] + b_ref[...], 0.0)
    o_ref[...] = jnp.dot(h, w_ref[...], preferred_element_type=jnp.float32) + fb_ref[...]

  return pl.pallas_call(
      body,
      out_shape=jax.ShapeDtypeStruct((n, d), jnp.float32),
  )(p, s, gamma.reshape(1, d), beta.reshape(1, d), fcT, fcb.reshape(1, d))


# speedup vs baseline: 4.5053x; 1.1861x over previous
import functools
import jax
import jax.numpy as jnp
import numpy as np
from jax import lax
from jax.experimental import pallas as pl
from jax.experimental.pallas import tpu as pltpu
from jax.experimental.pallas import tpu_sc as plsc

NC = 2
NS = 16
NW = NC * NS


def _unpack_perm(d):
  perm = np.zeros(d, dtype=np.int32)
  h = d // 2
  for cb in range(d // 32):
    for i in range(16):
      perm[cb * 16 + i] = cb * 32 + i
      perm[h + cb * 16 + i] = cb * 32 + 16 + i
  return perm


def _emb_gather(emb, xpad, n_pad, d):
  rows_w = n_pad // NW
  ch = 80
  nch = rows_w // ch
  mesh = plsc.VectorSubcoreMesh(core_axis_name="c", subcore_axis_name="s")

  @functools.partial(
      pl.kernel,
      out_type=jax.ShapeDtypeStruct((n_pad, d), jnp.float32),
      mesh=mesh,
      scratch_types=[
          pltpu.VMEM((ch,), jnp.int32),
          pltpu.VMEM((ch, d), jnp.float32),
          pltpu.SemaphoreType.DMA,
      ],
  )
  def gather_k(emb_hbm, x_hbm, out_hbm, idx_v, rows_v, sem):
    wid = lax.axis_index("s") * NC + lax.axis_index("c")
    base = wid * rows_w

    def body(j, carry):
      off = base + j * ch
      pltpu.sync_copy(x_hbm.at[pl.ds(off, ch)], idx_v)
      pltpu.async_copy(emb_hbm.at[idx_v], rows_v, sem).wait()
      pltpu.sync_copy(rows_v, out_hbm.at[pl.ds(off, ch)])
      return carry

    lax.fori_loop(0, nch, body, 0)

  return gather_k(emb, xpad)


def _edge_pass(K, QV, idxcat, n_pad, d, e_pad):
  c = 40
  nch_tot = e_pad // (NS * c)
  nf = 416
  ns = nch_tot - nf
  fast_cid = 0
  rows_acc = n_pad // NS
  mesh = plsc.VectorSubcoreMesh(core_axis_name="c", subcore_axis_name="s")
  zeros = jnp.zeros((rows_acc, d), jnp.float32)

  @functools.partial(
      pl.kernel,
      out_type=jax.ShapeDtypeStruct((NC, n_pad, d), jnp.float32),
      mesh=mesh,
      scratch_types=[
          pltpu.VMEM((4, 3, c), jnp.int32),
          pltpu.VMEM((2, c, d), jnp.float32),
          pltpu.VMEM((2, c, d), jnp.int32),
          pltpu.VMEM((2, c, d), jnp.float32),
          pltpu.VMEM_SHARED((n_pad, d), jnp.float32),
          pltpu.SemaphoreType.DMA,
          pltpu.SemaphoreType.DMA,
          pltpu.SemaphoreType.DMA,
          pltpu.SemaphoreType.DMA,
          pltpu.SemaphoreType.DMA,
          pltpu.SemaphoreType.DMA,
          pltpu.SemaphoreType.DMA,
      ],
  )
  def edge_k(k_hbm, qv_hbm, idx_hbm, z_hbm, out_hbm,
             ib, kv, qvv, mv, acc,
             sk0, sk1, sq0, sq1, ss0, ss1, si):
    cid = lax.axis_index("c")
    sid = lax.axis_index("s")
    is_fast = cid == fast_cid
    nch = jnp.where(is_fast, nf, ns)
    row0 = jnp.where(is_fast, sid * nf, NS * nf + sid * ns)
    sks = (sk0, sk1)
    sqs = (sq0, sq1)
    sss = (ss0, ss1)

    pltpu.sync_copy(z_hbm, acc.at[pl.ds(sid * rows_acc, rows_acc)])
    plsc.subcore_barrier()

    def fire(j, u, b):
      pltpu.async_copy(k_hbm.at[ib.at[u, 1]], kv.at[b], sks[b])
      pltpu.async_copy(qv_hbm.at[ib.at[u, 0]], qvv.at[b], sqs[b])

    for b in range(2):
      pltpu.sync_copy(idx_hbm.at[row0 + b], ib.at[b])
      fire(b, b, b)

    def quad(jo, carry):
      for u in range(4):
        b = u % 2
        u2 = (u + 2) % 4
        j = jo * 4 + u
        pltpu.make_async_copy(k_hbm.at[ib.at[u, 1]], kv.at[b], sks[b]).wait()
        pltpu.make_async_copy(qv_hbm.at[ib.at[u, 0]], qvv.at[b], sqs[b]).wait()

        @pl.when(j >= 2)
        def _():
          pltpu.make_async_copy(mv.at[b], acc.at[ib.at[u2, 2]], sss[b]).wait()

        @pl.when(j + 2 < nch)
        def _():
          pltpu.async_copy(idx_hbm.at[row0 + j + 2], ib.at[u2], si)

        himask = jnp.int32(-65536)
        hw = d // 2

        def wide(xi):
          e = jax.lax.bitcast_convert_type(xi << 16, jnp.float32)
          o = jax.lax.bitcast_convert_type(xi & himask, jnp.float32)
          return e, o

        def rowp(r2, rc):
          for rr in range(2):
            r = r2 * 2 + rr
            for cb in range(d // 32):
              sle = pl.ds(cb * 16, 16)
              slo = pl.ds(hw + cb * 16, 16)
              qe, qo = wide(qvv[b, r, sle])
              ve, vo = wide(qvv[b, r, slo])
              se = 1.0 / (1.0 + jnp.exp(kv[b, r, sle] + qe))
              so = 1.0 / (1.0 + jnp.exp(kv[b, r, slo] + qo))
              mv[b, r, pl.ds(cb * 32, 16)] = se * ve
              mv[b, r, pl.ds(cb * 32 + 16, 16)] = so * vo
          return rc
        lax.fori_loop(0, c // 2, rowp, 0)

        pltpu.async_copy(mv.at[b], acc.at[ib.at[u, 2]], sss[b], add=True)

        @pl.when(j + 2 < nch)
        def _():
          pltpu.make_async_copy(idx_hbm.at[row0 + j + 2], ib.at[u2], si).wait()
          fire(j + 2, u2, b)
      return carry

    lax.fori_loop(0, nch // 4, quad, 0)

    for b in range(2):
      pltpu.make_async_copy(mv.at[b], acc.at[ib.at[2 + b, 2]], sss[b]).wait()

    plsc.subcore_barrier()
    pltpu.sync_copy(acc.at[pl.ds(sid * rows_acc, rows_acc)],
                    out_hbm.at[cid, pl.ds(sid * rows_acc, rows_acc)])

  return edge_k(K, QV, idxcat, zeros)


def _pack_bf16(x, d):
  u = jax.lax.bitcast_convert_type(x.astype(jnp.bfloat16),
                                   jnp.uint16).astype(jnp.int32)
  return u[:, :d // 2] | (u[:, d // 2:] << 16)


def _projections(h, wkT, bk, wqT, bq, wvT, bv, wsT, bo):
  n, d = h.shape

  def body(h_ref, wk_r, bk_r, wq_r, bq_r, wv_r, bv_r, ws_r, bo_r,
           k_o, qv_o, s_o):
    hh = h_ref[...]
    k_o[...] = jnp.dot(hh, wk_r[...], preferred_element_type=jnp.float32) + bk_r[...]
    qp = _pack_bf16(
        jnp.dot(hh, wq_r[...], preferred_element_type=jnp.float32) + bq_r[...], d)
    vp = _pack_bf16(
        jnp.dot(hh, wv_r[...], preferred_element_type=jnp.float32) + bv_r[...], d)
    qv_o[...] = jnp.concatenate([qp, vp], axis=1)
    s_o[...] = jnp.dot(hh, ws_r[...], preferred_element_type=jnp.float32) + bo_r[...]

  return pl.pallas_call(
      body,
      out_shape=[jax.ShapeDtypeStruct((n, d), jnp.float32),
                 jax.ShapeDtypeStruct((n, d), jnp.int32),
                 jax.ShapeDtypeStruct((n, d), jnp.float32)],
  )(h, wkT, bk.reshape(1, d), wqT, bq.reshape(1, d), wvT, bv.reshape(1, d),
    wsT, bo.reshape(1, d))


def kernel(x, edge_index, emb, Wk1, bk1, Wq1, bq1, Wv1, bv1, Ws1, bo1,
           gamma1, beta1, Wk2, bk2, Wq2, bq2, Wv2, bv2, Ws2, bo2,
           gamma2, beta2, fcW, fcb):
  n, d = emb.shape
  e = edge_index.shape[1]
  n_pad = 10240
  n_acc = 10112
  e_pad = 327680

  x = x.astype(jnp.int32)
  src = edge_index[0].astype(jnp.int32)
  dst = edge_index[1].astype(jnp.int32)
  xpad = jnp.concatenate([x, jnp.zeros((n_pad - n,), jnp.int32)])
  srcp = jnp.concatenate([src, jnp.zeros((e_pad - e,), jnp.int32)])
  dstg = jnp.concatenate([dst, jnp.zeros((e_pad - e,), jnp.int32)])
  dsts = jnp.concatenate([dst, jnp.full((e_pad - e,), n, jnp.int32)])
  cc = 40
  idxcat = jnp.stack([srcp.reshape(-1, cc), dstg.reshape(-1, cc),
                      dsts.reshape(-1, cc)], axis=1)

  h = _emb_gather(emb, xpad, n_pad, d)[:n]

  neg = jnp.float32(-1.0)
  perm = _unpack_perm(d)
  wk1 = Wk1.T[:, perm] * neg
  wq1 = Wq1.T[:, perm] * neg
  wv1 = Wv1.T[:, perm]
  wk2 = Wk2.T[:, perm] * neg
  wq2 = Wq2.T[:, perm] * neg
  wv2 = Wv2.T[:, perm]
  k1, qv1, s1 = _projections(h, wk1, bk1[perm] * neg, wq1, bq1[perm] * neg,
                             wv1, bv1[perm], Ws1.T, bo1)
  p1 = _edge_pass(k1, qv1, idxcat, n_acc, d, e_pad)
  k2, qv2, s2 = _combine_bn_proj(p1, s1, gamma1, beta1,
                                 wk2, bk2[perm] * neg,
                                 wq2, bq2[perm] * neg, wv2, bv2[perm],
                                 Ws2.T, bo2)
  p2 = _edge_pass(k2, qv2, idxcat, n_acc, d, e_pad)
  return _final_affine(p2, s2, gamma2, beta2, fcW.T, fcb)


def _combine_bn_proj(p, s, gamma, beta, wkT, bk, wqT, bq, wvT, bv, wsT, bo):
  n, d = s.shape

  def body(p_ref, s_ref, g_ref, b_ref, wk_r, bk_r, wq_r, bq_r, wv_r, bv_r,
           ws_r, bo_r, k_o, qv_o, s_o):
    pre = p_ref[0, :n, :] + p_ref[1, :n, :] + s_ref[...]
    mu = jnp.sum(pre, axis=0, keepdims=True) * (1.0 / n)
    cen = pre - mu
    var = jnp.sum(cen * cen, axis=0, keepdims=True) * (1.0 / n)
    h = jnp.maximum(
        cen * lax.rsqrt(var + 1e-5) * g_ref[...] + b_ref[...], 0.0)
    k_o[...] = jnp.dot(h, wk_r[...], preferred_element_type=jnp.float32) + bk_r[...]
    qp = _pack_bf16(
        jnp.dot(h, wq_r[...], preferred_element_type=jnp.float32) + bq_r[...], d)
    vp = _pack_bf16(
        jnp.dot(h, wv_r[...], preferred_element_type=jnp.float32) + bv_r[...], d)
    qv_o[...] = jnp.concatenate([qp, vp], axis=1)
    s_o[...] = jnp.dot(h, ws_r[...], preferred_element_type=jnp.float32) + bo_r[...]

  return pl.pallas_call(
      body,
      out_shape=[jax.ShapeDtypeStruct((n, d), jnp.float32),
                 jax.ShapeDtypeStruct((n, d), jnp.int32),
                 jax.ShapeDtypeStruct((n, d), jnp.float32)],
  )(p, s, gamma.reshape(1, d), beta.reshape(1, d), wkT, bk.reshape(1, d),
    wqT, bq.reshape(1, d), wvT, bv.reshape(1, d), wsT, bo.reshape(1, d))


def _final_affine(p, s, gamma, beta, fcT, fcb):
  n, d = s.shape

  def body(p_ref, s_ref, g_ref, b_ref, w_ref, fb_ref, o_ref):
    pre = p_ref[0, :n, :] + p_ref[1, :n, :] + s_ref[...]
    mu = jnp.sum(pre, axis=0, keepdims=True) * (1.0 / n)
    cen = pre - mu
    var = jnp.sum(cen * cen, axis=0, keepdims=True) * (1.0 / n)
    h = jnp.maximum(
        cen * lax.rsqrt(var + 1e-5) * g_ref[...] + b_ref[...], 0.0)
    o_ref[...] = jnp.dot(h, w_ref[...], preferred_element_type=jnp.float32) + fb_ref[...]

  return pl.pallas_call(
      body,
      out_shape=jax.ShapeDtypeStruct((n, d), jnp.float32),
  )(p, s, gamma.reshape(1, d), beta.reshape(1, d), fcT, fcb.reshape(1, d))
